# baseline (speedup 1.0000x reference)
import functools

import jax
import jax.numpy as jnp
from jax import lax
from jax.experimental import pallas as pl
from jax.experimental.pallas import tpu as pltpu
from jax.experimental.pallas import tpu_sc as plsc

_B, _C, _T = 16384, 14, 200
_NB = 128


def _build_cmap_bcast(perm):
    p = perm.astype(jnp.int32)
    m = jnp.concatenate([
        p,
        jnp.array([6], jnp.int32),
        p + 7,
        jnp.array([13], jnp.int32),
    ])
    return jnp.broadcast_to(m[:, None], (_C, 16))


def kernel(chromatogram_batch, perm):
    x = chromatogram_batch
    cmap_bc = _build_cmap_bcast(perm)
    xf = x.reshape(_B * _C, _T)
    info = plsc.get_sparse_core_info()
    nc, ns = info.num_cores, info.num_subcores
    nw = nc * ns
    bw = _B // nw
    nchunks = bw // _NB
    mesh = plsc.VectorSubcoreMesh(core_axis_name="c", subcore_axis_name="s")

    @functools.partial(
        pl.kernel,
        mesh=mesh,
        out_type=jax.ShapeDtypeStruct((_B, _C, _T), jnp.float32),
        compiler_params=pltpu.CompilerParams(
            needs_layout_passes=False, use_tc_tiling_on_sc=False
        ),
        scratch_types=[
            pltpu.VMEM((_C, 16), jnp.int32),
            pltpu.VMEM((_NB,), jnp.int32),
            pltpu.VMEM((_NB, _T), jnp.float32),
            pltpu.SemaphoreType.DMA,
        ],
    )
    def k(xf_hbm, cmap_hbm, out_hbm, cmap_v, idx_v, buf_v, sem):
        wid = lax.axis_index("s") * nc + lax.axis_index("c")
        b0 = wid * bw
        pltpu.sync_copy(cmap_hbm, cmap_v)
        lane = lax.broadcasted_iota(jnp.int32, (16,), 0)
        for c in range(_C):
            mc = cmap_v[c]
            for j in range(nchunks):
                base = b0 + j * _NB
                for u in range(_NB // 16):
                    idx_v[pl.ds(u * 16, 16)] = (base + u * 16 + lane) * _C + mc
                pltpu.async_copy(xf_hbm.at[idx_v], buf_v, sem).wait()
                pltpu.sync_copy(buf_v, out_hbm.at[pl.ds(base, _NB), c])

    return k(xf, cmap_bc)
